# baseline (speedup 1.0000x reference)
import functools

import jax
import jax.numpy as jnp
from jax import lax
from jax.experimental import pallas as pl
from jax.experimental.pallas import tpu as pltpu
from jax.experimental.pallas import tpu_sc as plsc

N = 1048576
X_LIM, Y_LIM, Z_LIM = 511, 511, 255

NC, NS = 2, 16
NW = NC * NS
PW = N // NW
S = 8192
NSUB = PW // S
VPC = S // 16
C = 512
VCC = C // 16

_mesh = plsc.VectorSubcoreMesh(core_axis_name="c", subcore_axis_name="s")


@functools.partial(
    pl.kernel,
    mesh=_mesh,
    compiler_params=pltpu.CompilerParams(
        needs_layout_passes=False, use_tc_tiling_on_sc=False),
    out_type=jax.ShapeDtypeStruct((4 * N,), jnp.float32),
    scratch_types=[
        pltpu.VMEM((S,), jnp.float32),
        pltpu.VMEM((S,), jnp.float32),
        pltpu.VMEM((S,), jnp.float32),
        pltpu.VMEM((S + 16,), jnp.int32),
        pltpu.VMEM((S + 16,), jnp.int32),
        pltpu.VMEM((C, 16), jnp.float32),
        pltpu.VMEM((C, 16), jnp.float32),
        pltpu.VMEM((4 * S,), jnp.float32),
        pltpu.SemaphoreType.DMA,
        pltpu.SemaphoreType.DMA,
    ],
)
def _ct_gather(xyz_hbm, img_hbm, out_hbm, x_v, y_v, z_v, crow_v, cpk_v,
               rows_a, rows_b, out_v, sem_a, sem_b):
    wid = lax.axis_index("s") * NC + lax.axis_index("c")
    iota = lax.iota(jnp.int32, 16)
    ones16 = jnp.full((16,), 1.0, jnp.float32)
    zeros16 = jnp.full((16,), 0.0, jnp.float32)
    zeros16i = jnp.full((16,), 0, jnp.int32)
    base = wid * PW

    def _fill(g, c):
        out_v[pl.ds(g * 16, 16)] = ones16
        return c
    lax.fori_loop(0, (4 * S) // 16, _fill, 0)

    def _fill0(g, c):
        crow_v[pl.ds(g * 16, 16)] = zeros16i
        return c
    lax.fori_loop(0, (S + 16) // 16, _fill0, 0)

    def _sub(sub, c):
        sbase = base + sub * S
        pltpu.sync_copy(xyz_hbm.at[pl.ds(sbase, S)], x_v)
        pltpu.sync_copy(xyz_hbm.at[pl.ds(N + sbase, S)], y_v)
        pltpu.sync_copy(xyz_hbm.at[pl.ds(2 * N + sbase, S)], z_v)

        def _comp(g, off):
            x = x_v[pl.ds(g * 16, 16)]
            y = y_v[pl.ds(g * 16, 16)]
            z = z_v[pl.ds(g * 16, 16)]
            ix = ((x + 1.0) * 255.5).astype(jnp.int32)
            iy = ((y + 1.0) * 255.5).astype(jnp.int32)
            iz = ((z + 1.0) * 127.5).astype(jnp.int32)
            good = ((ix.astype(jnp.uint32) <= X_LIM)
                    & (iy.astype(jnp.uint32) <= Y_LIM)
                    & (iz.astype(jnp.uint32) <= Z_LIM))
            phys = ((ix << 17) + ((iy >> 3) << 11) + ((iz >> 7) << 10)
                    + ((iy & 7) << 7) + (iz & 127))
            plsc.store_compressed(crow_v.at[pl.ds(off, 16)], phys >> 4,
                                  mask=good)
            pk = ((g * 16 + iota) << 4) | (phys & 15)
            plsc.store_compressed(cpk_v.at[pl.ds(off, 16)], pk, mask=good)
            return off + jnp.max(plsc.all_reduce_population_count(good))
        n_valid = lax.fori_loop(0, VPC, _comp, jnp.int32(0))

        def _zero(g, cc):
            b = g * 16
            out_v[pl.ds((b >> 7) * 512 + 384 + (b & 127), 16)] = zeros16
            return cc
        lax.fori_loop(0, VPC, _zero, 0)

        nch = (n_valid + (C - 1)) // C

        def _fire(j, rows_v, sem):
            pltpu.async_copy(img_hbm.at[crow_v.at[pl.ds(j * C, C)]],
                             rows_v, sem)

        def _drain(j, rows_v, sem):
            pltpu.make_async_copy(img_hbm.at[crow_v.at[pl.ds(j * C, C)]],
                                  rows_v, sem).wait()

        def _extract(j, rows_v):
            def _e(g, cc):
                pk = cpk_v[pl.ds(j * C + g * 16, 16)]
                sv = plsc.load_gather(rows_v, [g * 16 + iota, pk & 15])
                pos = pk >> 4
                slot = ((pos >> 7) << 9) + 384 + (pos & 127)
                ok = (j * C + g * 16 + iota) < n_valid
                plsc.store_scatter(out_v, [slot], sv, mask=ok)
                return cc
            lax.fori_loop(0, VCC, _e, 0)

        @pl.when(nch > 0)
        def _():
            _fire(0, rows_a, sem_a)

        def _chunks(j2, cc):
            j = j2 * 2

            @pl.when(j + 1 < nch)
            def _():
                _fire(j + 1, rows_b, sem_b)
            _drain(j, rows_a, sem_a)
            _extract(j, rows_a)

            @pl.when(j + 2 < nch)
            def _():
                _fire(j + 2, rows_a, sem_a)

            @pl.when(j + 1 < nch)
            def _():
                _drain(j + 1, rows_b, sem_b)
                _extract(j + 1, rows_b)
            return cc
        lax.fori_loop(0, (nch + 1) // 2, _chunks, 0)

        pltpu.sync_copy(out_v, out_hbm.at[pl.ds(4 * sbase, 4 * S)])
        return c
    lax.fori_loop(0, NSUB, _sub, 0)


def kernel(xyz, img):
    xyz_planar = jnp.transpose(xyz, (2, 0, 1)).reshape(3 * N)
    img_rows = (img.reshape(512, 64, 8, 2, 128)
                .transpose(0, 1, 3, 2, 4).reshape(4 * N, 16))
    out = _ct_gather(xyz_planar, img_rows)
    return out.reshape(N // 128, 4, 128).transpose(0, 2, 1).reshape(1, N, 4)


# speedup vs baseline: 1.0012x; 1.0012x over previous
import functools

import jax
import jax.numpy as jnp
from jax import lax
from jax.experimental import pallas as pl
from jax.experimental.pallas import tpu as pltpu
from jax.experimental.pallas import tpu_sc as plsc

N = 1048576
X_LIM, Y_LIM, Z_LIM = 511, 511, 255

NC, NS = 2, 16
NW = NC * NS
PW = N // NW
S = 8192
NSUB = PW // S
VPC = S // 16
C = 512
PSTEP = 96

_mesh = plsc.VectorSubcoreMesh(core_axis_name="c", subcore_axis_name="s")


@functools.partial(
    pl.kernel,
    mesh=_mesh,
    compiler_params=pltpu.CompilerParams(needs_layout_passes=False),
    out_type=jax.ShapeDtypeStruct((4 * N,), jnp.float32),
    scratch_types=[
        pltpu.VMEM((S,), jnp.float32),
        pltpu.VMEM((S,), jnp.float32),
        pltpu.VMEM((S,), jnp.float32),
        pltpu.VMEM((S + 16,), jnp.int32),
        pltpu.VMEM((S + 16,), jnp.int32),
        pltpu.VMEM((S + 16,), jnp.int32),
        pltpu.VMEM((S + 16,), jnp.int32),
        pltpu.VMEM((S,), jnp.float32),
        pltpu.VMEM((S,), jnp.float32),
        pltpu.VMEM((4 * S,), jnp.float32),
        pltpu.SemaphoreType.DMA,
        pltpu.SemaphoreType.DMA,
    ],
)
def _ct_gather(xyz_hbm, img_hbm, out_hbm, x_v, y_v, z_v, cidx_a, cidx_b,
               cpos_a, cpos_b, sig_a, sig_b, out_v, sem_e, sem_o):
    wid = lax.axis_index("s") * NC + lax.axis_index("c")
    iota = lax.iota(jnp.int32, 16)
    ones16 = jnp.full((16,), 1.0, jnp.float32)
    zeros16 = jnp.full((16,), 0.0, jnp.float32)
    zeros16i = jnp.full((16,), 0, jnp.int32)
    base = wid * PW

    def _fill(g, c):
        out_v[pl.ds(g * 16, 16)] = ones16
        return c
    lax.fori_loop(0, (4 * S) // 16, _fill, 0)

    def _fill0(g, c):
        cidx_a[pl.ds(g * 16, 16)] = zeros16i
        cidx_b[pl.ds(g * 16, 16)] = zeros16i
        return c
    lax.fori_loop(0, (S + 16) // 16, _fill0, 0)

    def _load(sub):
        sbase = base + sub * S
        pltpu.sync_copy(xyz_hbm.at[pl.ds(sbase, S)], x_v)
        pltpu.sync_copy(xyz_hbm.at[pl.ds(N + sbase, S)], y_v)
        pltpu.sync_copy(xyz_hbm.at[pl.ds(2 * N + sbase, S)], z_v)

    def _p1_slice(cidx_v, cpos_v, p_off, hi):
        p, off = p_off

        def _comp(g, o):
            x = x_v[pl.ds(g * 16, 16)]
            y = y_v[pl.ds(g * 16, 16)]
            z = z_v[pl.ds(g * 16, 16)]
            ix = ((x + 1.0) * 255.5).astype(jnp.int32)
            iy = ((y + 1.0) * 255.5).astype(jnp.int32)
            iz = ((z + 1.0) * 127.5).astype(jnp.int32)
            good = ((ix.astype(jnp.uint32) <= X_LIM)
                    & (iy.astype(jnp.uint32) <= Y_LIM)
                    & (iz.astype(jnp.uint32) <= Z_LIM))
            phys = ((ix << 17) + ((iy >> 3) << 11) + ((iz >> 7) << 10)
                    + ((iy & 7) << 7) + (iz & 127))
            plsc.store_compressed(cidx_v.at[pl.ds(o, 16)], phys, mask=good)
            plsc.store_compressed(cpos_v.at[pl.ds(o, 16)], g * 16 + iota,
                                  mask=good)
            return o + jnp.max(plsc.all_reduce_population_count(good))
        lim = jnp.minimum(p + hi, VPC)
        off = lax.fori_loop(p, lim, _comp, off)
        return (lim, off)

    def _fire(cidx_v, sig_v, sem, j):
        pltpu.async_copy(img_hbm.at[cidx_v.at[pl.ds(j * C, C)]],
                         sig_v.at[pl.ds(j * C, C)], sem)

    def _drain(cidx_v, sig_v, sem, j):
        pltpu.make_async_copy(img_hbm.at[cidx_v.at[pl.ds(j * C, C)]],
                              sig_v.at[pl.ds(j * C, C)], sem).wait()

    def _extract(cpos_v, sig_v, n_valid, j):
        def _e(g, cc):
            k = j * (C // 16) + g
            sv = sig_v[pl.ds(k * 16, 16)]
            pos = cpos_v[pl.ds(k * 16, 16)]
            slot = ((pos >> 7) << 9) + 384 + (pos & 127)
            ok = (k * 16 + iota) < n_valid
            plsc.store_scatter(out_v, [slot], sv, mask=ok)
            return cc
        lax.fori_loop(0, C // 16, _e, 0)

    def _zero_sigma():
        def _zero(g, cc):
            b = g * 16
            out_v[pl.ds((b >> 7) * 512 + 384 + (b & 127), 16)] = zeros16
            return cc
        lax.fori_loop(0, VPC, _zero, 0)

    bufs = [(cidx_a, cpos_a, sig_a), (cidx_b, cpos_b, sig_b)]
    sems = [sem_e, sem_o]

    _load(0)
    _, nv0 = _p1_slice(cidx_a, cpos_a, (jnp.int32(0), jnp.int32(0)), VPC)
    nv = [None] * NSUB
    nv[0] = nv0

    for i in range(NSUB):
        ci, cp, sg = bufs[i % 2]
        cj, cq, sh = bufs[(i + 1) % 2]
        n_valid = nv[i]
        nch = (n_valid + (C - 1)) // C
        _zero_sigma()
        if i + 1 < NSUB:
            _load(i + 1)

        @pl.when(nch > 0)
        def _():
            _fire(ci, sg, sems[0], 0)

        def _chunks(j2, p_off, ci=ci, cp=cp, sg=sg, cj=cj, cq=cq,
                    n_valid=n_valid, nch=nch, last=(i + 1 == NSUB)):
            j = j2 * 2
            if not last:
                p_off = _p1_slice(cj, cq, p_off, PSTEP)
            _drain(ci, sg, sems[0], j)

            @pl.when(j + 1 < nch)
            def _():
                _fire(ci, sg, sems[1], j + 1)
            _extract(cp, sg, n_valid, j)
            if not last:
                p_off = _p1_slice(cj, cq, p_off, PSTEP)

            @pl.when(j + 1 < nch)
            def _():
                _drain(ci, sg, sems[1], j + 1)

            @pl.when(j + 2 < nch)
            def _():
                _fire(ci, sg, sems[0], j + 2)

            @pl.when(j + 1 < nch)
            def _():
                _extract(cp, sg, n_valid, j + 1)
            return p_off

        p_off = lax.fori_loop(0, (nch + 1) // 2, _chunks,
                              (jnp.int32(0), jnp.int32(0)))
        if i + 1 < NSUB:
            _, nv[i + 1] = _p1_slice(cj, cq, p_off, VPC)

        pltpu.sync_copy(out_v, out_hbm.at[pl.ds(4 * (base + i * S), 4 * S)])


def kernel(xyz, img):
    xyz_planar = jnp.transpose(xyz, (2, 0, 1)).reshape(3 * N)
    img_tiled = (img.reshape(512, 64, 8, 2, 128)
                 .transpose(0, 1, 3, 2, 4).reshape(64 * N))
    out = _ct_gather(xyz_planar, img_tiled)
    return out.reshape(N // 128, 4, 128).transpose(0, 2, 1).reshape(1, N, 4)


# speedup vs baseline: 1.0682x; 1.0668x over previous
import functools

import jax
import jax.numpy as jnp
from jax import lax
from jax.experimental import pallas as pl
from jax.experimental.pallas import tpu as pltpu
from jax.experimental.pallas import tpu_sc as plsc

N = 1048576
X_LIM, Y_LIM, Z_LIM = 511, 511, 255

NC, NS = 2, 16
NW = NC * NS
PW = N // NW
S = 8192
NSUB = PW // S
VPC = S // 16
C = 512

_mesh = plsc.VectorSubcoreMesh(core_axis_name="c", subcore_axis_name="s")


@functools.partial(
    pl.kernel,
    mesh=_mesh,
    compiler_params=pltpu.CompilerParams(needs_layout_passes=False),
    out_type=jax.ShapeDtypeStruct((4 * N,), jnp.float32),
    scratch_types=[
        pltpu.VMEM((S,), jnp.float32),
        pltpu.VMEM((S,), jnp.float32),
        pltpu.VMEM((S,), jnp.float32),
        pltpu.VMEM((S + 16,), jnp.int32),
        pltpu.VMEM((S + 16,), jnp.int32),
        pltpu.VMEM((S,), jnp.float32),
        pltpu.VMEM((4 * S,), jnp.float32),
        pltpu.SemaphoreType.DMA,
    ],
)
def _ct_gather(xyz_hbm, img_hbm, out_hbm, x_v, y_v, z_v, cidx_v, cpos_v,
               sig_v, out_v, sem):
    wid = lax.axis_index("s") * NC + lax.axis_index("c")
    iota = lax.iota(jnp.int32, 16)
    ones16 = jnp.full((16,), 1.0, jnp.float32)
    zeros16 = jnp.full((16,), 0.0, jnp.float32)
    zeros16i = jnp.full((16,), 0, jnp.int32)
    base = wid * PW

    def _fill(g, c):
        out_v[pl.ds(g * 16, 16)] = ones16
        return c
    lax.fori_loop(0, (4 * S) // 16, _fill, 0)

    def _fill0(g, c):
        cidx_v[pl.ds(g * 16, 16)] = zeros16i
        return c
    lax.fori_loop(0, (S + 16) // 16, _fill0, 0)

    def _sub(sub, c):
        sbase = base + sub * S
        pltpu.sync_copy(xyz_hbm.at[pl.ds(sbase, S)], x_v)
        pltpu.sync_copy(xyz_hbm.at[pl.ds(N + sbase, S)], y_v)
        pltpu.sync_copy(xyz_hbm.at[pl.ds(2 * N + sbase, S)], z_v)

        @plsc.parallel_loop(0, VPC, unroll=4,
                            carry=jnp.zeros((16,), jnp.int32))
        def _comp(g, off_vec):
            x = x_v[pl.ds(g * 16, 16)]
            y = y_v[pl.ds(g * 16, 16)]
            z = z_v[pl.ds(g * 16, 16)]
            ix = ((x + 1.0) * 255.5).astype(jnp.int32)
            iy = ((y + 1.0) * 255.5).astype(jnp.int32)
            iz = ((z + 1.0) * 127.5).astype(jnp.int32)
            good = ((ix.astype(jnp.uint32) <= X_LIM)
                    & (iy.astype(jnp.uint32) <= Y_LIM)
                    & (iz.astype(jnp.uint32) <= Z_LIM))
            phys = ((ix << 17) + ((iy >> 3) << 11) + ((iz >> 7) << 10)
                    + ((iy & 7) << 7) + (iz & 127))
            goodi = good.astype(jnp.int32)
            rank = jnp.cumsum(goodi) - goodi
            addr = off_vec + rank
            plsc.store_scatter(cidx_v, [addr], phys, mask=good)
            plsc.store_scatter(cpos_v, [addr], g * 16 + iota, mask=good)
            return off_vec + plsc.all_reduce_population_count(good)
        n_valid = jnp.max(_comp)

        @plsc.parallel_loop(0, VPC, unroll=4)
        def _zero(g):
            b = g * 16
            out_v[pl.ds((b >> 7) * 512 + 384 + (b & 127), 16)] = zeros16

        nch = (n_valid + (C - 1)) // C

        def _fire(j, cc):
            pltpu.async_copy(img_hbm.at[cidx_v.at[pl.ds(j * C, C)]],
                             sig_v.at[pl.ds(j * C, C)], sem)
            return cc
        lax.fori_loop(0, nch, _fire, 0)

        def _drain(j, cc):
            pltpu.make_async_copy(img_hbm.at[cidx_v.at[pl.ds(j * C, C)]],
                                  sig_v.at[pl.ds(j * C, C)], sem).wait()
            return cc
        lax.fori_loop(0, nch, _drain, 0)

        nvec = (n_valid + 15) >> 4

        @plsc.parallel_loop(0, nvec, unroll=2)
        def _outp(g):
            sv = sig_v[pl.ds(g * 16, 16)]
            pos = cpos_v[pl.ds(g * 16, 16)]
            slot = ((pos >> 7) << 9) + 384 + (pos & 127)
            ok = (g * 16 + iota) < n_valid
            plsc.store_scatter(out_v, [slot], sv, mask=ok)

        pltpu.sync_copy(out_v, out_hbm.at[pl.ds(4 * sbase, 4 * S)])
        return c
    lax.fori_loop(0, NSUB, _sub, 0)


def kernel(xyz, img):
    xyz_planar = jnp.transpose(xyz, (2, 0, 1)).reshape(3 * N)
    img_tiled = (img.reshape(512, 64, 8, 2, 128)
                 .transpose(0, 1, 3, 2, 4).reshape(64 * N))
    out = _ct_gather(xyz_planar, img_tiled)
    return out.reshape(N // 128, 4, 128).transpose(0, 2, 1).reshape(1, N, 4)
